# Initial kernel scaffold; baseline (speedup 1.0000x reference)
import jax
import jax.numpy as jnp
from jax.experimental import pallas as pl


def kernel(x_embed, wte):
    raise NotImplementedError("write your pallas kernel here")


# speedup vs baseline: 11.2861x; 11.2861x over previous
import jax
import jax.numpy as jnp
from jax.experimental import pallas as pl

_K = 500
_C = 768
_TS = 512


def _body(x_ref, wte_ref, out_e_ref, out_s_ref, out_i_ref, out_r_ref):
    w = wte_ref[...]
    w_sq = jnp.sum(w * w, axis=1, keepdims=True)
    wn = w * jax.lax.rsqrt(jnp.maximum(w_sq, 1e-12))

    x = x_ref[...]
    x_sq = jnp.sum(x * x, axis=1, keepdims=True)
    xn = x * jax.lax.rsqrt(jnp.maximum(x_sq, 1e-12))

    sims = jnp.dot(xn, wn.T, preferred_element_type=jnp.float32)
    out_s_ref[...] = sims

    m = jnp.max(sims, axis=1, keepdims=True)
    iota_k = jax.lax.broadcasted_iota(jnp.int32, sims.shape, 1)
    idx = jnp.min(jnp.where(sims == m, iota_k, _K), axis=1, keepdims=True)
    out_i_ref[...] = idx

    onehot = (iota_k == idx).astype(jnp.float32)
    sel = jnp.dot(onehot, w, preferred_element_type=jnp.float32)
    out_e_ref[...] = sel + x

    @pl.when(pl.program_id(0) == 0)
    def _init():
        out_r_ref[...] = jnp.zeros_like(out_r_ref)

    out_r_ref[...] += jnp.sum(m).reshape(1, 1)


def kernel(x_embed, wte):
    B, S, C = x_embed.shape
    n_tok = B * S
    x_flat = x_embed.reshape(n_tok, C)
    grid = (n_tok // _TS,)

    out_e, out_s, out_i, out_r = pl.pallas_call(
        _body,
        grid=grid,
        in_specs=[
            pl.BlockSpec((_TS, C), lambda i: (i, 0)),
            pl.BlockSpec((_K, C), lambda i: (0, 0)),
        ],
        out_specs=[
            pl.BlockSpec((_TS, C), lambda i: (i, 0)),
            pl.BlockSpec((_TS, _K), lambda i: (i, 0)),
            pl.BlockSpec((_TS, 1), lambda i: (i, 0)),
            pl.BlockSpec((1, 1), lambda i: (0, 0)),
        ],
        out_shape=[
            jax.ShapeDtypeStruct((n_tok, C), jnp.float32),
            jax.ShapeDtypeStruct((n_tok, _K), jnp.float32),
            jax.ShapeDtypeStruct((n_tok, 1), jnp.int32),
            jax.ShapeDtypeStruct((1, 1), jnp.float32),
        ],
    )(x_flat, wte)

    prompted_embedding = out_e.reshape(B, S, C)
    similarity = out_s.reshape(B, S, _K)
    idx = out_i.reshape(B, S, 1)
    reduce_sim = out_r[0, 0] / jnp.float32(B)
    return prompted_embedding, reduce_sim, similarity, idx
